# baseline (speedup 1.0000x reference)
import functools

import jax
import jax.numpy as jnp
from jax import lax
from jax.experimental import pallas as pl
from jax.experimental.pallas import tpu as pltpu
from jax.experimental.pallas import tpu_sc as plsc

N_SIDE = 3
CUBE = N_SIDE ** 3
CH = 16
CHUNK = 128
NWORKERS = 32
LANES = 16
H1, H2 = 128, 64
BN = 2048


def _sc_gather(vol3, i0, i1, i2, npad, pg, ch):
  nchunks = npad // (NWORKERS * CHUNK)
  pts_per_w = npad // NWORKERS
  row = ch
  mesh = plsc.VectorSubcoreMesh(core_axis_name="c", subcore_axis_name="s")

  @functools.partial(
      pl.kernel,
      mesh=mesh,
      compiler_params=pltpu.CompilerParams(use_tc_tiling_on_sc=False),
      out_type=jax.ShapeDtypeStruct((npad, N_SIDE, N_SIDE, N_SIDE, row),
                                    jnp.float32),
      scratch_types=[
          pltpu.VMEM((CHUNK,), jnp.int32),
          pltpu.VMEM((CHUNK,), jnp.int32),
          pltpu.VMEM((CHUNK,), jnp.int32),
          pltpu.VMEM((CHUNK, N_SIDE, N_SIDE, N_SIDE, row), jnp.float32),
          pltpu.SemaphoreType.DMA,
      ],
  )
  def k(vol_hbm, i0_hbm, i1_hbm, i2_hbm, x_hbm, s0, s1, s2, xbuf, gsem):
    wid = lax.axis_index("s") * 2 + lax.axis_index("c")

    def body(c, carry):
      pbase = wid * pts_per_w + c * CHUNK
      pltpu.sync_copy(i0_hbm.at[pl.ds(pbase, CHUNK)], s0)
      pltpu.sync_copy(i1_hbm.at[pl.ds(pbase, CHUNK)], s1)
      pltpu.sync_copy(i2_hbm.at[pl.ds(pbase, CHUNK)], s2)

      for g in range(CHUNK // LANES):
        a0 = s0[pl.ds(g * LANES, LANES)]
        a1 = s1[pl.ds(g * LANES, LANES)]
        a2 = s2[pl.ds(g * LANES, LANES)]
        for l in range(LANES):
          pltpu.async_copy(
              vol_hbm.at[pl.ds(a0[l], N_SIDE), pl.ds(a1[l], N_SIDE),
                         pl.ds(a2[l], N_SIDE), :],
              xbuf.at[g * LANES + l], gsem)
      pltpu.make_async_copy(x_hbm.at[pl.ds(0, CHUNK)], xbuf, gsem).wait()
      pltpu.sync_copy(xbuf, x_hbm.at[pl.ds(pbase, CHUNK)])
      return carry

    lax.fori_loop(0, nchunks, body, 0)

  return k(vol3, i0, i1, i2)


def _mlp_body(x_ref, w1_ref, b1_ref, w2_ref, b2_ref, w3_ref, b3_ref, o_ref):
  h = jnp.dot(x_ref[...], w1_ref[...], preferred_element_type=jnp.float32)
  h = jnp.maximum(h + b1_ref[...], 0.0)
  h = jnp.dot(h, w2_ref[...], preferred_element_type=jnp.float32)
  h = jnp.maximum(h + b2_ref[...], 0.0)
  t = jnp.dot(h, w3_ref[...], preferred_element_type=jnp.float32)
  o_ref[...] = jnp.tanh(t + b3_ref[...])


def _mlp(x, w1p, b1p, w2, b2, w3, b3, npad):
  in_dim = x.shape[1]
  return pl.pallas_call(
      _mlp_body,
      grid=(npad // BN,),
      in_specs=[
          pl.BlockSpec((BN, in_dim), lambda i: (i, 0)),
          pl.BlockSpec((in_dim, H1), lambda i: (0, 0)),
          pl.BlockSpec((1, H1), lambda i: (0, 0)),
          pl.BlockSpec((H1, H2), lambda i: (0, 0)),
          pl.BlockSpec((1, H2), lambda i: (0, 0)),
          pl.BlockSpec((H2, 1), lambda i: (0, 0)),
          pl.BlockSpec((1, 1), lambda i: (0, 0)),
      ],
      out_specs=pl.BlockSpec((BN, 1), lambda i: (i, 0)),
      out_shape=jax.ShapeDtypeStruct((npad, 1), jnp.float32),
  )(x, w1p, b1p, w2, b2, w3, b3)


def kernel(tsdf, weights, features, indices, W1, b1, W2, b2, W3, b3):
  g = tsdf.shape[0]
  pg = g + 2
  feat = features.shape[-1]
  n = indices.shape[0]
  step = NWORKERS * CHUNK
  npad = -(-n // step) * step

  ch = 2 + feat
  packed = jnp.concatenate(
      [tsdf[..., None] + 0.1, weights[..., None], features], axis=-1)
  vol = jnp.pad(packed, ((1, 1), (1, 1), (1, 1), (0, 0)))

  idx = jnp.pad(indices.astype(jnp.int32), ((0, npad - n), (0, 0)))
  i0, i1, i2 = idx[:, 0], idx[:, 1], idx[:, 2]

  x = _sc_gather(vol, i0, i1, i2, npad, pg, ch)
  x = x.reshape(npad, CUBE * ch)

  w1r = W1.reshape(CUBE, ch, H1)
  b1p = (b1 - 0.1 * jnp.sum(w1r[:, 0, :], axis=0)).reshape(1, H1)

  out = _mlp(x, W1, b1p, W2, b2.reshape(1, H2), W3, b3.reshape(1, 1), npad)
  return out[:n]
